# baseline (speedup 1.0000x reference)
import functools

import jax
import jax.numpy as jnp
from jax.experimental import pallas as pl
from jax.experimental.pallas import tpu as pltpu

_EMB = 512
_M = 2048
_NBUF = 4
_TRAIL = 2


def _gather_body(idx_ref, w_ref, out_ref, *bufs_and_sems, nsteps):
    bufs = bufs_and_sems[:_NBUF]
    gsem, wsem = bufs_and_sems[_NBUF:]
    k = pl.program_id(0)

    for p in range(_NBUF):
        buf = bufs[p]

        @pl.when(jnp.logical_and(k < nsteps, k % _NBUF == p))
        def _issue():
            @pl.when(k >= _NBUF)
            def _wait_write():
                pltpu.make_async_copy(buf, out_ref.at[pl.ds(0, _M)], wsem.at[p]).wait()

            base = k * _M
            for m in range(_M):
                row = idx_ref[base + m]
                pltpu.make_async_copy(
                    w_ref.at[pl.ds(row, 1)],
                    buf.at[pl.ds(m, 1)],
                    gsem.at[p],
                ).start()

        @pl.when(
            jnp.logical_and(
                jnp.logical_and(k >= _TRAIL, k < nsteps + _TRAIL),
                (k - _TRAIL) % _NBUF == p,
            )
        )
        def _flush_prev():
            pltpu.make_async_copy(w_ref.at[pl.ds(0, _M)], buf, gsem.at[p]).wait()
            pltpu.make_async_copy(
                buf, out_ref.at[pl.ds((k - _TRAIL) * _M, _M)], wsem.at[p]
            ).start()

    @pl.when(k == nsteps + _TRAIL)
    def _final():
        for p in range(_NBUF):
            pltpu.make_async_copy(bufs[p], out_ref.at[pl.ds(0, _M)], wsem.at[p]).wait()


def kernel(input, weight):
    bptt, bsize = input.shape
    n = bptt * bsize
    idx = input.reshape(n).astype(jnp.int32)
    nsteps = n // _M

    grid_spec = pltpu.PrefetchScalarGridSpec(
        num_scalar_prefetch=1,
        grid=(nsteps + _TRAIL + 1,),
        in_specs=[pl.BlockSpec(memory_space=pl.ANY)],
        out_specs=pl.BlockSpec(memory_space=pl.ANY),
        scratch_shapes=(
            [pltpu.VMEM((_M, _EMB), jnp.float32) for _ in range(_NBUF)]
            + [pltpu.SemaphoreType.DMA((_NBUF,)),
               pltpu.SemaphoreType.DMA((_NBUF,))]
        ),
    )
    out = pl.pallas_call(
        functools.partial(_gather_body, nsteps=nsteps),
        grid_spec=grid_spec,
        out_shape=jax.ShapeDtypeStruct((n, _EMB), jnp.float32),
        compiler_params=pltpu.CompilerParams(
            dimension_semantics=("arbitrary",),
            disable_bounds_checks=True,
        ),
    )(idx, weight)
    return out.reshape(bptt, bsize, _EMB)


# speedup vs baseline: 1.0311x; 1.0311x over previous
import functools

import jax
import jax.numpy as jnp
from jax.experimental import pallas as pl
from jax.experimental.pallas import tpu as pltpu

_EMB = 512
_M = 1024
_NBUF = 4
_TRAIL = 2


def _gather_body(idx_ref, w_ref, out_ref, *bufs_and_sems, nsteps):
    bufs = bufs_and_sems[:_NBUF]
    gsem, wsem = bufs_and_sems[_NBUF:]
    k = pl.program_id(0)

    for p in range(_NBUF):
        buf = bufs[p]

        @pl.when(jnp.logical_and(k < nsteps, k % _NBUF == p))
        def _issue():
            @pl.when(k >= _NBUF)
            def _wait_write():
                pltpu.make_async_copy(buf, out_ref.at[pl.ds(0, _M)], wsem.at[p]).wait()

            base = k * _M
            for m in range(_M):
                row = idx_ref[base + m]
                pltpu.make_async_copy(
                    w_ref.at[pl.ds(row, 1)],
                    buf.at[pl.ds(m, 1)],
                    gsem.at[p],
                ).start()

        @pl.when(
            jnp.logical_and(
                jnp.logical_and(k >= _TRAIL, k < nsteps + _TRAIL),
                (k - _TRAIL) % _NBUF == p,
            )
        )
        def _flush_prev():
            pltpu.make_async_copy(w_ref.at[pl.ds(0, _M)], buf, gsem.at[p]).wait()
            pltpu.make_async_copy(
                buf, out_ref.at[pl.ds((k - _TRAIL) * _M, _M)], wsem.at[p]
            ).start()

    @pl.when(k == nsteps + _TRAIL)
    def _final():
        for p in range(_NBUF):
            pltpu.make_async_copy(bufs[p], out_ref.at[pl.ds(0, _M)], wsem.at[p]).wait()


def kernel(input, weight):
    bptt, bsize = input.shape
    n = bptt * bsize
    idx = input.reshape(n).astype(jnp.int32)
    nsteps = n // _M

    grid_spec = pltpu.PrefetchScalarGridSpec(
        num_scalar_prefetch=1,
        grid=(nsteps + _TRAIL + 1,),
        in_specs=[pl.BlockSpec(memory_space=pl.ANY)],
        out_specs=pl.BlockSpec(memory_space=pl.ANY),
        scratch_shapes=(
            [pltpu.VMEM((_M, _EMB), jnp.float32) for _ in range(_NBUF)]
            + [pltpu.SemaphoreType.DMA((_NBUF,)),
               pltpu.SemaphoreType.DMA((_NBUF,))]
        ),
    )
    out = pl.pallas_call(
        functools.partial(_gather_body, nsteps=nsteps),
        grid_spec=grid_spec,
        out_shape=jax.ShapeDtypeStruct((n, _EMB), jnp.float32),
        compiler_params=pltpu.CompilerParams(
            dimension_semantics=("arbitrary",),
            disable_bounds_checks=True,
        ),
    )(idx, weight)
    return out.reshape(bptt, bsize, _EMB)
